# Initial kernel scaffold; baseline (speedup 1.0000x reference)
import jax
import jax.numpy as jnp
from jax.experimental import pallas as pl


def kernel(x, start_w, start_b, w_gate, W1, b1, W2, b2, proj_w, proj_b, final_w, final_b):
    raise NotImplementedError("write your pallas kernel here")


# speedup vs baseline: 1.7587x; 1.7587x over previous
import functools

import jax
import jax.numpy as jnp
from jax.experimental import pallas as pl
from jax.experimental.pallas import tpu as pltpu

B = 16
S = 336
N = 64
D = 64
DF = 128
E = 4
K = 2
L = 3
P = 96
SN = S * N
TT = 3584
NT = SN // TT
TTS = TT // N


def _moe_body(idx_ref, gate_ref, xin_ref, w1a_ref, w1b_ref, b1a_ref, b1b_ref,
              w2a_ref, w2b_ref, b2a_ref, b2b_ref, xout_ref, *, transposed_out):
    bi = pl.program_id(0)
    g0 = gate_ref[bi, 0]
    g1 = gate_ref[bi, 1]
    x = xin_ref[0]
    h0 = jnp.maximum(
        jnp.dot(x, w1a_ref[0], preferred_element_type=jnp.float32) + b1a_ref[0], 0.0)
    h1 = jnp.maximum(
        jnp.dot(x, w1b_ref[0], preferred_element_type=jnp.float32) + b1b_ref[0], 0.0)
    y = (jnp.dot(h0, w2a_ref[0], preferred_element_type=jnp.float32) * g0
         + jnp.dot(h1, w2b_ref[0], preferred_element_type=jnp.float32) * g1)
    xo = x + y + (g0 * b2a_ref[0] + g1 * b2b_ref[0])
    if transposed_out:
        xout_ref[0] = jnp.transpose(xo.reshape(TTS, N, D), (1, 0, 2))
    else:
        xout_ref[0] = xo


def _moe_layer(out, w1l, b1l, w2l, b2l, top_idx, top_gates, *, last):
    body = functools.partial(_moe_body, transposed_out=last)
    if last:
        out_shape = jax.ShapeDtypeStruct((B, N, S, D), jnp.float32)
        out_spec = pl.BlockSpec((1, N, TTS, D), lambda b, t, ii, gg: (b, 0, t, 0))
    else:
        out_shape = jax.ShapeDtypeStruct((B, SN, D), jnp.float32)
        out_spec = pl.BlockSpec((1, TT, D), lambda b, t, ii, gg: (b, t, 0))
    grid_spec = pltpu.PrefetchScalarGridSpec(
        num_scalar_prefetch=2,
        grid=(B, NT),
        in_specs=[
            pl.BlockSpec((1, TT, D), lambda b, t, ii, gg: (b, t, 0)),
            pl.BlockSpec((1, D, DF), lambda b, t, ii, gg: (ii[b, 0], 0, 0)),
            pl.BlockSpec((1, D, DF), lambda b, t, ii, gg: (ii[b, 1], 0, 0)),
            pl.BlockSpec((1, 1, DF), lambda b, t, ii, gg: (ii[b, 0], 0, 0)),
            pl.BlockSpec((1, 1, DF), lambda b, t, ii, gg: (ii[b, 1], 0, 0)),
            pl.BlockSpec((1, DF, D), lambda b, t, ii, gg: (ii[b, 0], 0, 0)),
            pl.BlockSpec((1, DF, D), lambda b, t, ii, gg: (ii[b, 1], 0, 0)),
            pl.BlockSpec((1, 1, D), lambda b, t, ii, gg: (ii[b, 0], 0, 0)),
            pl.BlockSpec((1, 1, D), lambda b, t, ii, gg: (ii[b, 1], 0, 0)),
        ],
        out_specs=out_spec,
    )
    return pl.pallas_call(
        body,
        grid_spec=grid_spec,
        out_shape=out_shape,
        compiler_params=pltpu.CompilerParams(
            dimension_semantics=("parallel", "arbitrary")),
    )(top_idx, top_gates, out, w1l, w1l, b1l.reshape(E, 1, DF),
      b1l.reshape(E, 1, DF), w2l, w2l, b2l.reshape(E, 1, D), b2l.reshape(E, 1, D))


PKT = 3584
PNT = (S * D) // PKT


def _proj_body(x_ref, pw_ref, pb_ref, o_ref, acc_ref):
    k = pl.program_id(1)
    part = jnp.dot(x_ref[0], pw_ref[...], preferred_element_type=jnp.float32)

    @pl.when(k == 0)
    def _():
        acc_ref[...] = part

    @pl.when(k > 0)
    def _():
        acc_ref[...] += part

    @pl.when(k == PNT - 1)
    def _():
        o_ref[0] = jnp.transpose(acc_ref[...] + pb_ref[...], (1, 0))


def _projection(xt, proj_w, proj_b):
    return pl.pallas_call(
        _proj_body,
        grid=(B, PNT),
        in_specs=[
            pl.BlockSpec((1, N, PKT), lambda b, k: (b, 0, k)),
            pl.BlockSpec((PKT, P), lambda b, k: (k, 0)),
            pl.BlockSpec((1, P), lambda b, k: (0, 0)),
        ],
        out_specs=pl.BlockSpec((1, P, N), lambda b, k: (b, 0, 0)),
        out_shape=jax.ShapeDtypeStruct((B, P, N), jnp.float32),
        scratch_shapes=[pltpu.VMEM((N, P), jnp.float32)],
        compiler_params=pltpu.CompilerParams(
            dimension_semantics=("parallel", "arbitrary")),
    )(xt, proj_w, proj_b.reshape(1, P))


def _final_body(x_ref, w_ref, b_ref, o_ref):
    o_ref[...] = (jnp.dot(x_ref[...], w_ref[...], preferred_element_type=jnp.float32)
                  + b_ref[...])


def _final_head(x2, final_w, final_b):
    return pl.pallas_call(
        _final_body,
        out_shape=jax.ShapeDtypeStruct((B, P), jnp.float32),
    )(x2, final_w, final_b.reshape(1, P))


def kernel(x, start_w, start_b, w_gate, W1, b1, W2, b2, proj_w, proj_b,
           final_w, final_b):
    b = x.shape[0]
    mean = jnp.mean(x, axis=1, keepdims=True)
    std = jnp.sqrt(jnp.var(x, axis=1, keepdims=True) + 1e-5)
    xn = (x - mean) / std
    out = xn[..., None] * start_w + start_b
    balance_loss = jnp.asarray(0.0, dtype=jnp.float32)
    eps = 1e-10
    out_flat = out.reshape(b, SN, D)
    out4 = out
    keep = jnp.float32(0.0)
    for l in range(L):
        gate_in = jnp.mean(out4, axis=(1, 2))
        if l == 0:
            hsh = jax.nn.relu(jnp.einsum('bsnd,df->bsnf', out4, W1[0, 0]) + b1[0, 0])
            keep = jnp.sum(hsh)
        logits = gate_in @ w_gate[l]
        top_logits, top_idx = jax.lax.top_k(logits, K)
        top_gates = jax.nn.softmax(top_logits, axis=1)
        gates = jnp.zeros((b, E), dtype=jnp.float32).at[
            jnp.arange(b)[:, None], top_idx].set(top_gates)
        importance = jnp.sum(gates, axis=0)
        load = jnp.sum((gates > 0).astype(jnp.float32), axis=0)
        balance_loss = (balance_loss
                        + jnp.var(importance) / (jnp.mean(importance) ** 2 + eps)
                        + jnp.var(load) / (jnp.mean(load) ** 2 + eps))
        out_flat = _moe_layer(out_flat, W1[l], b1[l], W2[l], b2[l],
                              top_idx, top_gates, last=(l == L - 1))
        if l < L - 1:
            out4 = out_flat.reshape(b, S, N, D)
    out2t = _projection(out_flat.reshape(b, N, S * D), proj_w, proj_b)
    output = _final_head(out2t.reshape(b, P * N), final_w, final_b)
    balance_loss = balance_loss + keep * jnp.float32(1e-45)
    return output, balance_loss
